# Initial kernel scaffold; baseline (speedup 1.0000x reference)
import jax
import jax.numpy as jnp
from jax.experimental import pallas as pl


def kernel(inputs, table):
    raise NotImplementedError("write your pallas kernel here")


# speedup vs baseline: 3.4288x; 3.4288x over previous
import jax
import jax.numpy as jnp
from jax.experimental import pallas as pl


def _copy_body(table_ref, out_ref):
    out_ref[...] = table_ref[...][None]


def kernel(inputs, table):
    B = inputs.shape[0]
    S, D = table.shape
    BS = 512
    grid = (S // BS, B)
    out = pl.pallas_call(
        _copy_body,
        grid=grid,
        in_specs=[pl.BlockSpec((BS, D), lambda s, b: (s, 0))],
        out_specs=pl.BlockSpec((1, BS, D), lambda s, b: (b, s, 0)),
        out_shape=jax.ShapeDtypeStruct((B, S, D), table.dtype),
    )(table)
    return out


# speedup vs baseline: 4.2069x; 1.2269x over previous
import jax
import jax.numpy as jnp
from jax.experimental import pallas as pl


def _copy_body(table_ref, out_ref):
    out_ref[...] = table_ref[...][None]


def kernel(inputs, table):
    B = inputs.shape[0]
    S, D = table.shape
    BS = 1024
    grid = (S // BS, B)
    out = pl.pallas_call(
        _copy_body,
        grid=grid,
        in_specs=[pl.BlockSpec((BS, D), lambda s, b: (s, 0))],
        out_specs=pl.BlockSpec((1, BS, D), lambda s, b: (b, s, 0)),
        out_shape=jax.ShapeDtypeStruct((B, S, D), table.dtype),
    )(table)
    return out


# speedup vs baseline: 4.6401x; 1.1030x over previous
import jax
import jax.numpy as jnp
from jax.experimental import pallas as pl


def _copy_body(table_ref, out_ref):
    out_ref[...] = table_ref[...][None]


def kernel(inputs, table):
    B = inputs.shape[0]
    S, D = table.shape
    BS = 2048
    grid = (S // BS, B)
    out = pl.pallas_call(
        _copy_body,
        grid=grid,
        in_specs=[pl.BlockSpec((BS, D), lambda s, b: (s, 0))],
        out_specs=pl.BlockSpec((1, BS, D), lambda s, b: (b, s, 0)),
        out_shape=jax.ShapeDtypeStruct((B, S, D), table.dtype),
    )(table)
    return out


# speedup vs baseline: 5.1318x; 1.1060x over previous
import jax
import jax.numpy as jnp
from jax.experimental import pallas as pl
from jax.experimental.pallas import tpu as pltpu


def _copy_body(table_ref, out_ref):
    blk = table_ref[...][None]
    out_ref[...] = jnp.broadcast_to(blk, out_ref.shape)


def kernel(inputs, table):
    B = inputs.shape[0]
    S, D = table.shape
    BS = 2048
    BB = 2
    grid = (S // BS, B // BB)
    out = pl.pallas_call(
        _copy_body,
        grid=grid,
        in_specs=[pl.BlockSpec((BS, D), lambda s, b: (s, 0))],
        out_specs=pl.BlockSpec((BB, BS, D), lambda s, b: (b, s, 0)),
        out_shape=jax.ShapeDtypeStruct((B, S, D), table.dtype),
        compiler_params=pltpu.CompilerParams(
            vmem_limit_bytes=128 * 1024 * 1024,
        ),
    )(table)
    return out


# speedup vs baseline: 5.1762x; 1.0087x over previous
import jax
import jax.numpy as jnp
from jax.experimental import pallas as pl
from jax.experimental.pallas import tpu as pltpu


def _copy_body(table_ref, out_ref):
    blk = table_ref[...][None]
    out_ref[...] = jnp.broadcast_to(blk, out_ref.shape)


def kernel(inputs, table):
    B = inputs.shape[0]
    S, D = table.shape
    BS = 1024
    BB = 4
    grid = (S // BS, B // BB)
    out = pl.pallas_call(
        _copy_body,
        grid=grid,
        in_specs=[pl.BlockSpec((BS, D), lambda s, b: (s, 0))],
        out_specs=pl.BlockSpec((BB, BS, D), lambda s, b: (b, s, 0)),
        out_shape=jax.ShapeDtypeStruct((B, S, D), table.dtype),
        compiler_params=pltpu.CompilerParams(
            vmem_limit_bytes=128 * 1024 * 1024,
        ),
    )(table)
    return out
